# Initial kernel scaffold; baseline (speedup 1.0000x reference)
import jax
import jax.numpy as jnp
from jax import lax
from jax.experimental import pallas as pl
from jax.experimental.pallas import tpu as pltpu
from jax.experimental.pallas import tpu_sc as plsc

N_NODES = 10000
EMB = 128
NUM_LAYER = 5
NUM_GRAPHS = 512
BN_EPS = 1e-5

NC = 2
NS = 16
NW = NC * NS

NPAD = 10240
PAD_ROW = N_NODES
NBLK = NPAD // 128

E_TOTAL = 320000
CK = 128
NCH = 80
EP = NW * NCH * CK
ROWS_PER_TILE = NPAD // NS

PG = 544
PBR = PG // NS
BK = 64
BPT = NPAD // NW
NBCH = BPT // BK


def _sc_mesh():
    return plsc.VectorSubcoreMesh(
        core_axis_name="c", subcore_axis_name="s",
        num_cores=NC, num_subcores=NS)


def _sc_degree(dst3, ones_c16, zeros_c16):
    def body(dst_hbm, ones_hbm, zeros_hbm, out_hbm, idx_v, ones_v, zero_v,
             accum):
        c = lax.axis_index("c")
        s = lax.axis_index("s")
        wid = s * NC + c
        pltpu.sync_copy(ones_hbm, ones_v)
        pltpu.sync_copy(zeros_hbm, zero_v)
        pltpu.sync_copy(dst_hbm.at[wid], idx_v)
        base = s * ROWS_PER_TILE
        for r in range(ROWS_PER_TILE // CK):
            pltpu.sync_copy(zero_v, accum.at[pl.ds(base + r * CK, CK)])
        plsc.subcore_barrier()

        def chunk(j, carry):
            pltpu.sync_copy(ones_v, accum.at[idx_v.at[j]], add=True)
            return carry

        lax.fori_loop(0, NCH, chunk, 0)
        plsc.subcore_barrier()
        pltpu.sync_copy(accum.at[pl.ds(base, ROWS_PER_TILE)],
                        out_hbm.at[c, pl.ds(base, ROWS_PER_TILE)])

    return pl.kernel(
        body,
        out_type=jax.ShapeDtypeStruct((NC, NPAD, 16), jnp.float32),
        mesh=_sc_mesh(),
        scratch_types=[
            pltpu.VMEM((NCH, CK), jnp.int32),
            pltpu.VMEM((CK, 16), jnp.float32),
            pltpu.VMEM((CK, 16), jnp.float32),
            pltpu.VMEM_SHARED((NPAD, 16), jnp.float32),
        ],
    )(dst3, ones_c16, zeros_c16)


def _sc_scatter(g, src3, dst3, zeros_ck):
    def body(g_hbm, src_hbm, dst_hbm, z_hbm, out_hbm, sidx, didx, buf0, buf1,
             accum, sem):
        c = lax.axis_index("c")
        s = lax.axis_index("s")
        wid = s * NC + c
        pltpu.sync_copy(z_hbm, buf0)
        base = s * ROWS_PER_TILE
        for r in range(ROWS_PER_TILE // CK):
            pltpu.sync_copy(buf0, accum.at[pl.ds(base + r * CK, CK)])
        pltpu.sync_copy(src_hbm.at[wid], sidx)
        pltpu.sync_copy(dst_hbm.at[wid], didx)
        plsc.subcore_barrier()

        pltpu.async_copy(g_hbm.at[sidx.at[0]], buf0, sem)

        def pair(j, carry):
            e = 2 * j
            pltpu.make_async_copy(g_hbm.at[sidx.at[e]], buf0, sem).wait()
            pltpu.async_copy(g_hbm.at[sidx.at[e + 1]], buf1, sem)
            pltpu.sync_copy(buf0, accum.at[didx.at[e]], add=True)
            pltpu.make_async_copy(g_hbm.at[sidx.at[e + 1]], buf1, sem).wait()

            @pl.when(j < NCH // 2 - 1)
            def _():
                pltpu.async_copy(g_hbm.at[sidx.at[e + 2]], buf0, sem)

            pltpu.sync_copy(buf1, accum.at[didx.at[e + 1]], add=True)
            return carry

        lax.fori_loop(0, NCH // 2, pair, 0)
        plsc.subcore_barrier()
        pltpu.sync_copy(accum.at[pl.ds(base, ROWS_PER_TILE)],
                        out_hbm.at[c, pl.ds(base, ROWS_PER_TILE)])

    return pl.kernel(
        body,
        out_type=jax.ShapeDtypeStruct((NC, NPAD, EMB), jnp.float32),
        mesh=_sc_mesh(),
        scratch_types=[
            pltpu.VMEM((NCH, CK), jnp.int32),
            pltpu.VMEM((NCH, CK), jnp.int32),
            pltpu.VMEM((CK, EMB), jnp.float32),
            pltpu.VMEM((CK, EMB), jnp.float32),
            pltpu.VMEM_SHARED((NPAD, EMB), jnp.float32),
            pltpu.SemaphoreType.DMA,
        ],
    )(g, src3, dst3, zeros_ck)


def _sc_pool(h, batch3, ones_b16, zeros_p, zeros_p16):
    def body(h_hbm, b_hbm, ones_hbm, zp_hbm, zc_hbm, psum_hbm, cnt_hbm,
             bidx, rows_v, ones_v, zp_v, zc_v, paccum, caccum):
        c = lax.axis_index("c")
        s = lax.axis_index("s")
        wid = s * NC + c
        pltpu.sync_copy(ones_hbm, ones_v)
        pltpu.sync_copy(zp_hbm, zp_v)
        pltpu.sync_copy(zc_hbm, zc_v)
        pltpu.sync_copy(b_hbm.at[wid], bidx)
        pltpu.sync_copy(zp_v, paccum.at[pl.ds(s * PBR, PBR)])
        pltpu.sync_copy(zc_v, caccum.at[pl.ds(s * PBR, PBR)])
        plsc.subcore_barrier()

        def chunk(t, carry):
            pltpu.sync_copy(h_hbm.at[pl.ds(wid * BPT + t * BK, BK)], rows_v)
            pltpu.sync_copy(rows_v, paccum.at[bidx.at[t]], add=True)
            pltpu.sync_copy(ones_v, caccum.at[bidx.at[t]], add=True)
            return carry

        lax.fori_loop(0, NBCH, chunk, 0)
        plsc.subcore_barrier()
        pltpu.sync_copy(paccum.at[pl.ds(s * PBR, PBR)],
                        psum_hbm.at[c, pl.ds(s * PBR, PBR)])
        pltpu.sync_copy(caccum.at[pl.ds(s * PBR, PBR)],
                        cnt_hbm.at[c, pl.ds(s * PBR, PBR)])

    return pl.kernel(
        body,
        out_type=(jax.ShapeDtypeStruct((NC, PG, EMB), jnp.float32),
                  jax.ShapeDtypeStruct((NC, PG, 16), jnp.float32)),
        mesh=_sc_mesh(),
        scratch_types=[
            pltpu.VMEM((NBCH, BK), jnp.int32),
            pltpu.VMEM((BK, EMB), jnp.float32),
            pltpu.VMEM((BK, 16), jnp.float32),
            pltpu.VMEM((PBR, EMB), jnp.float32),
            pltpu.VMEM((PBR, 16), jnp.float32),
            pltpu.VMEM_SHARED((PG, EMB), jnp.float32),
            pltpu.VMEM_SHARED((PG, 16), jnp.float32),
        ],
    )(h, batch3, ones_b16, zeros_p, zeros_p16)


def _tc_dinv(degp):
    def body(d0_ref, d1_ref, o_ref):
        i = pl.program_id(0)
        deg = 1.0 + d0_ref[0, :, 0:1] + d1_ref[0, :, 0:1]
        dinv = lax.rsqrt(deg)
        rid = i * 128 + lax.broadcasted_iota(jnp.int32, (128, 1), 0)
        dinv = jnp.where(rid < N_NODES, dinv, 0.0)
        o_ref[...] = jnp.broadcast_to(dinv, (128, EMB))

    return pl.pallas_call(
        body,
        grid=(NBLK,),
        in_specs=[pl.BlockSpec((1, 128, 16), lambda i: (0, i, 0)),
                  pl.BlockSpec((1, 128, 16), lambda i: (1, i, 0))],
        out_specs=pl.BlockSpec((128, EMB), lambda i: (i, 0)),
        out_shape=jax.ShapeDtypeStruct((NPAD, EMB), jnp.float32),
    )(degp, degp)


def _tc_entry(x, w, bias, dinvb):
    def body(x_ref, w_ref, b_ref, dv_ref, o_ref):
        h = jnp.dot(x_ref[...], w_ref[...], preferred_element_type=jnp.float32)
        o_ref[...] = dv_ref[...] * (h + b_ref[...])

    return pl.pallas_call(
        body,
        grid=(NBLK,),
        in_specs=[pl.BlockSpec((128, EMB), lambda i: (i, 0)),
                  pl.BlockSpec((EMB, EMB), lambda i: (0, 0)),
                  pl.BlockSpec((1, EMB), lambda i: (0, 0)),
                  pl.BlockSpec((128, EMB), lambda i: (i, 0))],
        out_specs=pl.BlockSpec((128, EMB), lambda i: (i, 0)),
        out_shape=jax.ShapeDtypeStruct((NPAD, EMB), jnp.float32),
    )(x, w, bias, dinvb)


def _tc_agg_stats(sp, g, dinvb):
    def body(s0_ref, s1_ref, g_ref, dv_ref, agg_ref, sum_ref, ssq_ref):
        i = pl.program_id(0)
        agg = dv_ref[...] * (s0_ref[0] + s1_ref[0] + g_ref[...])
        agg_ref[...] = agg

        @pl.when(i == 0)
        def _():
            sum_ref[...] = jnp.zeros_like(sum_ref)
            ssq_ref[...] = jnp.zeros_like(ssq_ref)

        sum_ref[...] += jnp.broadcast_to(
            jnp.sum(agg, axis=0, keepdims=True), (8, EMB))
        ssq_ref[...] += jnp.broadcast_to(
            jnp.sum(agg * agg, axis=0, keepdims=True), (8, EMB))

    return pl.pallas_call(
        body,
        grid=(NBLK,),
        in_specs=[pl.BlockSpec((1, 128, EMB), lambda i: (0, i, 0)),
                  pl.BlockSpec((1, 128, EMB), lambda i: (1, i, 0)),
                  pl.BlockSpec((128, EMB), lambda i: (i, 0)),
                  pl.BlockSpec((128, EMB), lambda i: (i, 0))],
        out_specs=(pl.BlockSpec((128, EMB), lambda i: (i, 0)),
                   pl.BlockSpec((8, EMB), lambda i: (0, 0)),
                   pl.BlockSpec((8, EMB), lambda i: (0, 0))),
        out_shape=(jax.ShapeDtypeStruct((NPAD, EMB), jnp.float32),
                   jax.ShapeDtypeStruct((8, EMB), jnp.float32),
                   jax.ShapeDtypeStruct((8, EMB), jnp.float32)),
    )(sp, sp, g, dinvb)


def _tc_mid(agg, ssum, ssq, gam, bet, w, bias, dinvb):
    def body(agg_ref, sum_ref, ssq_ref, gam_ref, bet_ref, w_ref, b_ref,
             dv_ref, o_ref):
        inv_n = 1.0 / N_NODES
        mean = sum_ref[0:1, :] * inv_n
        var = ssq_ref[0:1, :] * inv_n - mean * mean
        a = gam_ref[...] * lax.rsqrt(var + BN_EPS)
        csh = bet_ref[...] - mean * a
        u = jnp.maximum(agg_ref[...] * a + csh, 0.0)
        h = jnp.dot(u, w_ref[...], preferred_element_type=jnp.float32)
        o_ref[...] = dv_ref[...] * (h + b_ref[...])

    return pl.pallas_call(
        body,
        grid=(NBLK,),
        in_specs=[pl.BlockSpec((128, EMB), lambda i: (i, 0)),
                  pl.BlockSpec((8, EMB), lambda i: (0, 0)),
                  pl.BlockSpec((8, EMB), lambda i: (0, 0)),
                  pl.BlockSpec((1, EMB), lambda i: (0, 0)),
                  pl.BlockSpec((1, EMB), lambda i: (0, 0)),
                  pl.BlockSpec((EMB, EMB), lambda i: (0, 0)),
                  pl.BlockSpec((1, EMB), lambda i: (0, 0)),
                  pl.BlockSpec((128, EMB), lambda i: (i, 0))],
        out_specs=pl.BlockSpec((128, EMB), lambda i: (i, 0)),
        out_shape=jax.ShapeDtypeStruct((NPAD, EMB), jnp.float32),
    )(agg, ssum, ssq, gam, bet, w, bias, dinvb)


def _tc_last(agg, ssum, ssq, gam, bet):
    def body(agg_ref, sum_ref, ssq_ref, gam_ref, bet_ref, o_ref):
        inv_n = 1.0 / N_NODES
        mean = sum_ref[0:1, :] * inv_n
        var = ssq_ref[0:1, :] * inv_n - mean * mean
        a = gam_ref[...] * lax.rsqrt(var + BN_EPS)
        csh = bet_ref[...] - mean * a
        o_ref[...] = agg_ref[...] * a + csh

    return pl.pallas_call(
        body,
        grid=(NBLK,),
        in_specs=[pl.BlockSpec((128, EMB), lambda i: (i, 0)),
                  pl.BlockSpec((8, EMB), lambda i: (0, 0)),
                  pl.BlockSpec((8, EMB), lambda i: (0, 0)),
                  pl.BlockSpec((1, EMB), lambda i: (0, 0)),
                  pl.BlockSpec((1, EMB), lambda i: (0, 0))],
        out_specs=pl.BlockSpec((128, EMB), lambda i: (i, 0)),
        out_shape=jax.ShapeDtypeStruct((NPAD, EMB), jnp.float32),
    )(agg, ssum, ssq, gam, bet)


def _tc_pool_div(psum, cnt):
    def body(p0_ref, p1_ref, c0_ref, c1_ref, o_ref):
        cc = c0_ref[0, :, 0:1] + c1_ref[0, :, 0:1]
        o_ref[...] = (p0_ref[0] + p1_ref[0]) / jnp.maximum(cc, 1.0)

    return pl.pallas_call(
        body,
        grid=(1,),
        in_specs=[pl.BlockSpec((1, NUM_GRAPHS, EMB), lambda i: (0, 0, 0)),
                  pl.BlockSpec((1, NUM_GRAPHS, EMB), lambda i: (1, 0, 0)),
                  pl.BlockSpec((1, NUM_GRAPHS, 16), lambda i: (0, 0, 0)),
                  pl.BlockSpec((1, NUM_GRAPHS, 16), lambda i: (1, 0, 0))],
        out_specs=pl.BlockSpec((NUM_GRAPHS, EMB), lambda i: (0, 0)),
        out_shape=jax.ShapeDtypeStruct((NUM_GRAPHS, EMB), jnp.float32),
    )(psum, psum, cnt, cnt)


def kernel(x, edge_index, edge_attr, batch, W, b, gamma, beta):
    del edge_attr
    f32 = jnp.float32

    src = edge_index[0].astype(jnp.int32)
    dst = edge_index[1].astype(jnp.int32)
    pad_e = jnp.full((EP - E_TOTAL,), PAD_ROW, dtype=jnp.int32)
    src3 = jnp.concatenate([src, pad_e]).reshape(NW, NCH, CK)
    dst3 = jnp.concatenate([dst, pad_e]).reshape(NW, NCH, CK)
    batch3 = jnp.concatenate(
        [batch.astype(jnp.int32),
         jnp.full((NPAD - N_NODES,), NUM_GRAPHS, dtype=jnp.int32)]
    ).reshape(NW, NBCH, BK)
    x_pad = jnp.concatenate(
        [x.astype(f32), jnp.zeros((NPAD - N_NODES, EMB), f32)], axis=0)

    ones_c16 = jnp.ones((CK, 16), f32)
    zeros_c16 = jnp.zeros((CK, 16), f32)
    zeros_ck = jnp.zeros((CK, EMB), f32)
    ones_b16 = jnp.ones((BK, 16), f32)
    zeros_p = jnp.zeros((PBR, EMB), f32)
    zeros_p16 = jnp.zeros((PBR, 16), f32)

    degp = _sc_degree(dst3, ones_c16, zeros_c16)
    dinvb = _tc_dinv(degp)
    g = _tc_entry(x_pad, W[0], b[0].reshape(1, EMB), dinvb)

    h_final = None
    for l in range(NUM_LAYER):
        sp = _sc_scatter(g, src3, dst3, zeros_ck)
        agg, ssum, ssq = _tc_agg_stats(sp, g, dinvb)
        gam = gamma[l].reshape(1, EMB)
        bet = beta[l].reshape(1, EMB)
        if l < NUM_LAYER - 1:
            g = _tc_mid(agg, ssum, ssq, gam, bet,
                        W[l + 1], b[l + 1].reshape(1, EMB), dinvb)
        else:
            h_final = _tc_last(agg, ssum, ssq, gam, bet)

    psum, cnt = _sc_pool(h_final, batch3, ones_b16, zeros_p, zeros_p16)
    return _tc_pool_div(psum, cnt)


# speedup vs baseline: 5.5857x; 5.5857x over previous
import jax
import jax.numpy as jnp
from jax import lax
from jax.experimental import pallas as pl
from jax.experimental.pallas import tpu as pltpu
from jax.experimental.pallas import tpu_sc as plsc

N_NODES = 10000
EMB = 128
NUM_LAYER = 5
NUM_GRAPHS = 512
BN_EPS = 1e-5

NC = 2
NS = 16
NW = NC * NS

NPAD = 10240
PAD_ROW = N_NODES
NBLK = NPAD // 128

E_TOTAL = 320000
CK = 128
NCH = 80
EP = NW * NCH * CK
ROWS_PER_TILE = NPAD // NS
ACC_R = 10112
ACC_PT = ACC_R // NS

PG = 640
PBR = PG // NS
BK = 64
BPT = NPAD // NW
NBCH = BPT // BK


def _sc_mesh():
    return plsc.VectorSubcoreMesh(
        core_axis_name="c", subcore_axis_name="s",
        num_cores=NC, num_subcores=NS)


def _fill_2d(ref, rows, cols, value):
    v = jnp.full((16,), value, jnp.float32)
    cpr = cols // 16

    def st(t, carry):
        ref[t // cpr, pl.ds((t % cpr) * 16, 16)] = v
        return carry

    lax.fori_loop(0, rows * cpr, st, 0)


def _unpack_chunk(packed, sidx_c, didx_c, k, r):
    def st(t, carry):
        v = packed[k, pl.ds(t * 16, 16)]
        sidx_c[r, pl.ds(t * 16, 16)] = lax.bitwise_and(v, 16383)
        didx_c[r, pl.ds(t * 16, 16)] = lax.shift_right_logical(v, 14)
        return carry

    lax.fori_loop(0, CK // 16, st, 0)


def _sc_degree(epack):
    def body(e_hbm, out_hbm, idx_v, ones_v, zero_v, accum):
        c = lax.axis_index("c")
        s = lax.axis_index("s")
        wid = s * NC + c
        _fill_2d(ones_v, CK, 16, 1.0)
        _fill_2d(zero_v, CK, 16, 0.0)
        pltpu.sync_copy(e_hbm.at[wid], idx_v)
        cpr = CK // 16

        def st(t, carry):
            r = t // cpr
            o = (t % cpr) * 16
            idx_v[r, pl.ds(o, 16)] = lax.shift_right_logical(
                idx_v[r, pl.ds(o, 16)], 14)
            return carry

        lax.fori_loop(0, NCH * cpr, st, 0)
        base = s * ROWS_PER_TILE
        for r in range(ROWS_PER_TILE // CK):
            pltpu.sync_copy(zero_v, accum.at[pl.ds(base + r * CK, CK)])
        plsc.subcore_barrier()

        def chunk(j, carry):
            pltpu.sync_copy(ones_v, accum.at[idx_v.at[j]], add=True)
            return carry

        lax.fori_loop(0, NCH, chunk, 0)
        plsc.subcore_barrier()
        pltpu.sync_copy(accum.at[pl.ds(base, ROWS_PER_TILE)],
                        out_hbm.at[c, pl.ds(base, ROWS_PER_TILE)])

    return pl.kernel(
        body,
        out_type=jax.ShapeDtypeStruct((NC, NPAD, 16), jnp.float32),
        mesh=_sc_mesh(),
        scratch_types=[
            pltpu.VMEM((NCH, CK), jnp.int32),
            pltpu.VMEM((CK, 16), jnp.float32),
            pltpu.VMEM((CK, 16), jnp.float32),
            pltpu.VMEM_SHARED((NPAD, 16), jnp.float32),
        ],
    )(epack)


def _sc_scatter(g, epack):
    def body(g_hbm, e_hbm, out_hbm, packed, sidx_c, didx_c, bufs, accum,
             sem):
        c = lax.axis_index("c")
        s = lax.axis_index("s")
        wid = s * NC + c
        pltpu.sync_copy(e_hbm.at[wid], packed)

        def zr(t, carry):
            bufs[0, t // 8, pl.ds((t % 8) * 16, 16)] = jnp.zeros(
                (16,), jnp.float32)
            return carry

        lax.fori_loop(0, CK * 8, zr, 0)
        base = s * ACC_PT
        for r in range(ACC_PT // CK):
            pltpu.sync_copy(bufs.at[0], accum.at[pl.ds(base + r * CK, CK)])
        rem = ACC_PT % CK
        pltpu.sync_copy(bufs.at[0, pl.ds(0, rem)],
                        accum.at[pl.ds(base + ACC_PT - rem, rem)])
        plsc.subcore_barrier()

        _unpack_chunk(packed, sidx_c, didx_c, 0, 0)
        pltpu.async_copy(g_hbm.at[sidx_c.at[0]], bufs.at[0], sem)

        def pair(j, carry):
            e = 2 * j
            _unpack_chunk(packed, sidx_c, didx_c, e + 1, 1)
            pltpu.make_async_copy(
                g_hbm.at[sidx_c.at[0]], bufs.at[0], sem).wait()
            pltpu.async_copy(g_hbm.at[sidx_c.at[1]], bufs.at[1], sem)
            pltpu.sync_copy(bufs.at[0], accum.at[didx_c.at[0]], add=True)
            _unpack_chunk(packed, sidx_c, didx_c, (e + 2) % NCH, 0)
            pltpu.make_async_copy(
                g_hbm.at[sidx_c.at[1]], bufs.at[1], sem).wait()

            @pl.when(j < NCH // 2 - 1)
            def _():
                pltpu.async_copy(g_hbm.at[sidx_c.at[0]], bufs.at[0], sem)

            pltpu.sync_copy(bufs.at[1], accum.at[didx_c.at[1]], add=True)
            return carry

        lax.fori_loop(0, NCH // 2, pair, 0)
        plsc.subcore_barrier()
        pltpu.sync_copy(accum.at[pl.ds(base, ACC_PT)],
                        out_hbm.at[c, pl.ds(base, ACC_PT)])

    return pl.kernel(
        body,
        out_type=jax.ShapeDtypeStruct((NC, NPAD, EMB), jnp.float32),
        mesh=_sc_mesh(),
        scratch_types=[
            pltpu.VMEM((NCH, CK), jnp.int32),
            pltpu.VMEM((2, CK), jnp.int32),
            pltpu.VMEM((2, CK), jnp.int32),
            pltpu.VMEM((2, CK, EMB), jnp.float32),
            pltpu.VMEM_SHARED((ACC_R, EMB), jnp.float32),
            pltpu.SemaphoreType.DMA,
        ],
    )(g, epack)


def _sc_pool(h, batch3):
    def body(h_hbm, b_hbm, psum_hbm, cnt_hbm,
             bidx, rows_v, ones_v, zp_v, zc_v, paccum, caccum):
        c = lax.axis_index("c")
        s = lax.axis_index("s")
        wid = s * NC + c
        _fill_2d(ones_v, BK, 16, 1.0)
        _fill_2d(zp_v, PBR, EMB, 0.0)
        _fill_2d(zc_v, PBR, 16, 0.0)
        pltpu.sync_copy(b_hbm.at[wid], bidx)
        pltpu.sync_copy(zp_v, paccum.at[pl.ds(s * PBR, PBR)])
        pltpu.sync_copy(zc_v, caccum.at[pl.ds(s * PBR, PBR)])
        plsc.subcore_barrier()

        def chunk(t, carry):
            pltpu.sync_copy(h_hbm.at[pl.ds(wid * BPT + t * BK, BK)], rows_v)
            pltpu.sync_copy(rows_v, paccum.at[bidx.at[t]], add=True)
            pltpu.sync_copy(ones_v, caccum.at[bidx.at[t]], add=True)
            return carry

        lax.fori_loop(0, NBCH, chunk, 0)
        plsc.subcore_barrier()
        pltpu.sync_copy(paccum.at[pl.ds(s * PBR, PBR)],
                        psum_hbm.at[c, pl.ds(s * PBR, PBR)])
        pltpu.sync_copy(caccum.at[pl.ds(s * PBR, PBR)],
                        cnt_hbm.at[c, pl.ds(s * PBR, PBR)])

    return pl.kernel(
        body,
        out_type=(jax.ShapeDtypeStruct((NC, PG, EMB), jnp.float32),
                  jax.ShapeDtypeStruct((NC, PG, 16), jnp.float32)),
        mesh=_sc_mesh(),
        scratch_types=[
            pltpu.VMEM((NBCH, BK), jnp.int32),
            pltpu.VMEM((BK, EMB), jnp.float32),
            pltpu.VMEM((BK, 16), jnp.float32),
            pltpu.VMEM((PBR, EMB), jnp.float32),
            pltpu.VMEM((PBR, 16), jnp.float32),
            pltpu.VMEM_SHARED((PG, EMB), jnp.float32),
            pltpu.VMEM_SHARED((PG, 16), jnp.float32),
        ],
    )(h, batch3)


def _tc_dinv(degp):
    def body(d0_ref, d1_ref, o_ref):
        i = pl.program_id(0)
        deg = 1.0 + d0_ref[0, :, 0:1] + d1_ref[0, :, 0:1]
        dinv = lax.rsqrt(deg)
        rid = i * 128 + lax.broadcasted_iota(jnp.int32, (128, 1), 0)
        dinv = jnp.where(rid < N_NODES, dinv, 0.0)
        o_ref[...] = jnp.broadcast_to(dinv, (128, EMB))

    return pl.pallas_call(
        body,
        grid=(NBLK,),
        in_specs=[pl.BlockSpec((1, 128, 16), lambda i: (0, i, 0)),
                  pl.BlockSpec((1, 128, 16), lambda i: (1, i, 0))],
        out_specs=pl.BlockSpec((128, EMB), lambda i: (i, 0)),
        out_shape=jax.ShapeDtypeStruct((NPAD, EMB), jnp.float32),
    )(degp, degp)


def _tc_entry(x, w, bias, dinvb):
    def body(x_ref, w_ref, b_ref, dv_ref, o_ref):
        h = jnp.dot(x_ref[...], w_ref[...], preferred_element_type=jnp.float32)
        o_ref[...] = dv_ref[...] * (h + b_ref[...])

    return pl.pallas_call(
        body,
        grid=(NBLK,),
        in_specs=[pl.BlockSpec((128, EMB), lambda i: (i, 0)),
                  pl.BlockSpec((EMB, EMB), lambda i: (0, 0)),
                  pl.BlockSpec((1, EMB), lambda i: (0, 0)),
                  pl.BlockSpec((128, EMB), lambda i: (i, 0))],
        out_specs=pl.BlockSpec((128, EMB), lambda i: (i, 0)),
        out_shape=jax.ShapeDtypeStruct((NPAD, EMB), jnp.float32),
    )(x, w, bias, dinvb)


def _tc_agg_stats(sp, g, dinvb):
    def body(s0_ref, s1_ref, g_ref, dv_ref, agg_ref, sum_ref, ssq_ref):
        i = pl.program_id(0)
        agg = dv_ref[...] * (s0_ref[0] + s1_ref[0] + g_ref[...])
        rid = i * 128 + lax.broadcasted_iota(jnp.int32, (128, 1), 0)
        agg = jnp.where(rid < N_NODES, agg, 0.0)
        agg_ref[...] = agg

        @pl.when(i == 0)
        def _():
            sum_ref[...] = jnp.zeros_like(sum_ref)
            ssq_ref[...] = jnp.zeros_like(ssq_ref)

        sum_ref[...] += jnp.broadcast_to(
            jnp.sum(agg, axis=0, keepdims=True), (8, EMB))
        ssq_ref[...] += jnp.broadcast_to(
            jnp.sum(agg * agg, axis=0, keepdims=True), (8, EMB))

    return pl.pallas_call(
        body,
        grid=(NBLK,),
        in_specs=[pl.BlockSpec((1, 128, EMB), lambda i: (0, i, 0)),
                  pl.BlockSpec((1, 128, EMB), lambda i: (1, i, 0)),
                  pl.BlockSpec((128, EMB), lambda i: (i, 0)),
                  pl.BlockSpec((128, EMB), lambda i: (i, 0))],
        out_specs=(pl.BlockSpec((128, EMB), lambda i: (i, 0)),
                   pl.BlockSpec((8, EMB), lambda i: (0, 0)),
                   pl.BlockSpec((8, EMB), lambda i: (0, 0))),
        out_shape=(jax.ShapeDtypeStruct((NPAD, EMB), jnp.float32),
                   jax.ShapeDtypeStruct((8, EMB), jnp.float32),
                   jax.ShapeDtypeStruct((8, EMB), jnp.float32)),
    )(sp, sp, g, dinvb)


def _tc_mid(agg, ssum, ssq, gam, bet, w, bias, dinvb):
    def body(agg_ref, sum_ref, ssq_ref, gam_ref, bet_ref, w_ref, b_ref,
             dv_ref, o_ref):
        inv_n = 1.0 / N_NODES
        mean = sum_ref[0:1, :] * inv_n
        var = ssq_ref[0:1, :] * inv_n - mean * mean
        a = gam_ref[...] * lax.rsqrt(var + BN_EPS)
        csh = bet_ref[...] - mean * a
        u = jnp.maximum(agg_ref[...] * a + csh, 0.0)
        h = jnp.dot(u, w_ref[...], preferred_element_type=jnp.float32)
        o_ref[...] = dv_ref[...] * (h + b_ref[...])

    return pl.pallas_call(
        body,
        grid=(NBLK,),
        in_specs=[pl.BlockSpec((128, EMB), lambda i: (i, 0)),
                  pl.BlockSpec((8, EMB), lambda i: (0, 0)),
                  pl.BlockSpec((8, EMB), lambda i: (0, 0)),
                  pl.BlockSpec((1, EMB), lambda i: (0, 0)),
                  pl.BlockSpec((1, EMB), lambda i: (0, 0)),
                  pl.BlockSpec((EMB, EMB), lambda i: (0, 0)),
                  pl.BlockSpec((1, EMB), lambda i: (0, 0)),
                  pl.BlockSpec((128, EMB), lambda i: (i, 0))],
        out_specs=pl.BlockSpec((128, EMB), lambda i: (i, 0)),
        out_shape=jax.ShapeDtypeStruct((NPAD, EMB), jnp.float32),
    )(agg, ssum, ssq, gam, bet, w, bias, dinvb)


def _tc_last(agg, ssum, ssq, gam, bet):
    def body(agg_ref, sum_ref, ssq_ref, gam_ref, bet_ref, o_ref):
        inv_n = 1.0 / N_NODES
        mean = sum_ref[0:1, :] * inv_n
        var = ssq_ref[0:1, :] * inv_n - mean * mean
        a = gam_ref[...] * lax.rsqrt(var + BN_EPS)
        csh = bet_ref[...] - mean * a
        o_ref[...] = agg_ref[...] * a + csh

    return pl.pallas_call(
        body,
        grid=(NBLK,),
        in_specs=[pl.BlockSpec((128, EMB), lambda i: (i, 0)),
                  pl.BlockSpec((8, EMB), lambda i: (0, 0)),
                  pl.BlockSpec((8, EMB), lambda i: (0, 0)),
                  pl.BlockSpec((1, EMB), lambda i: (0, 0)),
                  pl.BlockSpec((1, EMB), lambda i: (0, 0))],
        out_specs=pl.BlockSpec((128, EMB), lambda i: (i, 0)),
        out_shape=jax.ShapeDtypeStruct((NPAD, EMB), jnp.float32),
    )(agg, ssum, ssq, gam, bet)


def _tc_pool_div(psum, cnt):
    def body(p0_ref, p1_ref, c0_ref, c1_ref, o_ref):
        cc = c0_ref[0, :, 0:1] + c1_ref[0, :, 0:1]
        o_ref[...] = (p0_ref[0] + p1_ref[0]) / jnp.maximum(cc, 1.0)

    return pl.pallas_call(
        body,
        grid=(1,),
        in_specs=[pl.BlockSpec((1, NUM_GRAPHS, EMB), lambda i: (0, 0, 0)),
                  pl.BlockSpec((1, NUM_GRAPHS, EMB), lambda i: (1, 0, 0)),
                  pl.BlockSpec((1, NUM_GRAPHS, 16), lambda i: (0, 0, 0)),
                  pl.BlockSpec((1, NUM_GRAPHS, 16), lambda i: (1, 0, 0))],
        out_specs=pl.BlockSpec((NUM_GRAPHS, EMB), lambda i: (0, 0)),
        out_shape=jax.ShapeDtypeStruct((NUM_GRAPHS, EMB), jnp.float32),
    )(psum, psum, cnt, cnt)


def kernel(x, edge_index, edge_attr, batch, W, b, gamma, beta):
    del edge_attr
    f32 = jnp.float32

    src = edge_index[0].astype(jnp.int32)
    dst = edge_index[1].astype(jnp.int32)
    packed = src + dst * 16384
    pad_e = jnp.full((EP - E_TOTAL,), PAD_ROW + PAD_ROW * 16384,
                     dtype=jnp.int32)
    epack = jnp.concatenate([packed, pad_e]).reshape(NW, NCH, CK)
    batch3 = jnp.concatenate(
        [batch.astype(jnp.int32),
         jnp.full((NPAD - N_NODES,), NUM_GRAPHS, dtype=jnp.int32)]
    ).reshape(NW, NBCH, BK)
    x_pad = jnp.concatenate(
        [x.astype(f32), jnp.zeros((NPAD - N_NODES, EMB), f32)], axis=0)

    degp = _sc_degree(epack)
    dinvb = _tc_dinv(degp)
    g = _tc_entry(x_pad, W[0], b[0].reshape(1, EMB), dinvb)

    h_final = None
    for l in range(NUM_LAYER):
        sp = _sc_scatter(g, epack)
        agg, ssum, ssq = _tc_agg_stats(sp, g, dinvb)
        gam = gamma[l].reshape(1, EMB)
        bet = beta[l].reshape(1, EMB)
        if l < NUM_LAYER - 1:
            g = _tc_mid(agg, ssum, ssq, gam, bet,
                        W[l + 1], b[l + 1].reshape(1, EMB), dinvb)
        else:
            h_final = _tc_last(agg, ssum, ssq, gam, bet)

    psum, cnt = _sc_pool(h_final, batch3)
    return _tc_pool_div(psum, cnt)
